# Initial kernel scaffold; baseline (speedup 1.0000x reference)
import jax
import jax.numpy as jnp
from jax.experimental import pallas as pl


def kernel(tree_h, W1, b1, W2, b2, segment_ids):
    raise NotImplementedError("write your pallas kernel here")


# speedup vs baseline: 2.7812x; 2.7812x over previous
import functools

import jax
import jax.numpy as jnp
from jax import lax
from jax.experimental import pallas as pl
from jax.experimental.pallas import tpu as pltpu
from jax.experimental.pallas import tpu_sc as plsc

_N_NODES = 32768
_DIM = 64
_NUM_SEG = 16
_NC = 2
_NS = 16
_NW = _NC * _NS
_ROWS_PER_W = _N_NODES // _NW
_WORDS_PER_W = _ROWS_PER_W * _DIM

_mesh = plsc.VectorSubcoreMesh(core_axis_name="c", subcore_axis_name="s")


@functools.partial(
    pl.kernel,
    out_type=(
        jax.ShapeDtypeStruct((_NW * _NUM_SEG * _DIM,), jnp.float32),
        jax.ShapeDtypeStruct((_NW * _NUM_SEG,), jnp.float32),
    ),
    mesh=_mesh,
    compiler_params=pltpu.CompilerParams(needs_layout_passes=False),
    scratch_types=(
        pltpu.VMEM((_WORDS_PER_W,), jnp.float32),
        pltpu.VMEM((_ROWS_PER_W,), jnp.int32),
        pltpu.VMEM((_NUM_SEG * _DIM,), jnp.float32),
        pltpu.VMEM((_NUM_SEG,), jnp.float32),
        pltpu.VMEM((2 * _NUM_SEG,), jnp.int32),
    ),
)
def _sc_segment_sums(tree_hbm, seg_hbm, sums_hbm, counts_hbm,
                     rows_v, seg_v, acc_v, cnt_v, bnd_v):
    wid = lax.axis_index("s") * _NC + lax.axis_index("c")
    row0 = wid * _ROWS_PER_W
    pltpu.sync_copy(seg_hbm.at[pl.ds(row0, _ROWS_PER_W)], seg_v)
    pltpu.sync_copy(tree_hbm.at[pl.ds(row0 * _DIM, _WORDS_PER_W)], rows_v)

    sval = lax.iota(jnp.int32, 16) + 1
    lo = jnp.zeros((16,), jnp.int32)
    hi = jnp.full((16,), _ROWS_PER_W, jnp.int32)
    for _ in range(11):
        active = lo < hi
        mid = jnp.minimum((lo + hi) // 2, _ROWS_PER_W - 1)
        below = plsc.load_gather(seg_v, [mid]) < sval
        lo = jnp.where(active & below, mid + 1, lo)
        hi = jnp.where(active & (~below), mid, hi)
    ub = lo

    bnd_v[pl.ds(0, 16)] = jnp.zeros((16,), jnp.int32)
    bnd_v[pl.ds(16, 16)] = ub
    ub_prev = bnd_v[pl.ds(15, 16)]
    cnt_v[pl.ds(0, 16)] = (ub - ub_prev).astype(jnp.float32)

    bounds = ([jnp.int32(0)] + [ub[s] for s in range(_NUM_SEG - 1)]
              + [jnp.int32(_ROWS_PER_W)])

    zero = jnp.zeros((16,), jnp.float32)
    for s in range(_NUM_SEG):
        start, end = bounds[s], bounds[s + 1]

        def body(r, accs):
            base = r * _DIM
            return tuple(a + rows_v[pl.ds(base + 16 * d, 16)]
                         for d, a in enumerate(accs))

        a0, a1, a2, a3 = lax.fori_loop(start, end, body,
                                       (zero, zero, zero, zero))
        acc_v[pl.ds(s * _DIM, 16)] = a0
        acc_v[pl.ds(s * _DIM + 16, 16)] = a1
        acc_v[pl.ds(s * _DIM + 32, 16)] = a2
        acc_v[pl.ds(s * _DIM + 48, 16)] = a3

    pltpu.sync_copy(acc_v, sums_hbm.at[pl.ds(wid * _NUM_SEG * _DIM,
                                             _NUM_SEG * _DIM)])
    pltpu.sync_copy(cnt_v, counts_hbm.at[pl.ds(wid * _NUM_SEG, _NUM_SEG)])


def _mlp_body(sums_ref, counts_ref, w1_ref, b1_ref, w2t_ref, b2_ref, out_ref):
    totals = jnp.sum(sums_ref[...], axis=0)
    counts = jnp.sum(counts_ref[...], axis=0)
    agg = totals / jnp.maximum(counts, 1.0)[:, None]
    h = lax.dot_general(agg, w1_ref[...], (((1,), (0,)), ((), ())),
                        preferred_element_type=jnp.float32)
    h = jnp.maximum(h + b1_ref[...], 0.0)
    out_ref[...] = jnp.sum(h * w2t_ref[...], axis=1, keepdims=True) + b2_ref[...]


_mlp = pl.pallas_call(
    _mlp_body,
    out_shape=jax.ShapeDtypeStruct((_NUM_SEG, 1), jnp.float32),
)


def kernel(tree_h, W1, b1, W2, b2, segment_ids):
    seg = segment_ids.astype(jnp.int32)
    sums_flat, counts_flat = _sc_segment_sums(tree_h.reshape(-1), seg)
    return _mlp(sums_flat.reshape(_NW, _NUM_SEG, _DIM),
                counts_flat.reshape(_NW, _NUM_SEG),
                W1, b1.reshape(1, _DIM), W2.reshape(1, _DIM),
                b2.reshape(1, 1))


# speedup vs baseline: 3.2645x; 1.1738x over previous
import functools

import jax
import jax.numpy as jnp
from jax import lax
from jax.experimental import pallas as pl
from jax.experimental.pallas import tpu as pltpu
from jax.experimental.pallas import tpu_sc as plsc

_N_NODES = 32768
_DIM = 64
_NUM_SEG = 16
_NC = 2
_NS = 16
_NW = _NC * _NS
_ROWS_PER_W = _N_NODES // _NW
_CHUNK = 512
_NCHUNK = _ROWS_PER_W // _CHUNK

_mesh = plsc.VectorSubcoreMesh(core_axis_name="c", subcore_axis_name="s")


@functools.partial(
    pl.kernel,
    out_type=(
        jax.ShapeDtypeStruct((_NW * _NUM_SEG * _DIM,), jnp.float32),
        jax.ShapeDtypeStruct((_NW * _NUM_SEG,), jnp.float32),
    ),
    mesh=_mesh,
    compiler_params=pltpu.CompilerParams(needs_layout_passes=False),
    scratch_types=(
        pltpu.VMEM((_CHUNK, _DIM), jnp.float32),
        pltpu.VMEM((_ROWS_PER_W,), jnp.int32),
        pltpu.VMEM((_NUM_SEG * _DIM,), jnp.float32),
        pltpu.VMEM((_NUM_SEG,), jnp.float32),
        pltpu.VMEM((2 * _NUM_SEG,), jnp.int32),
    ),
)
def _sc_segment_sums(tree_hbm, seg_hbm, sums_hbm, counts_hbm,
                     rows_v, seg_v, acc_v, cnt_v, bnd_v):
    wid = lax.axis_index("s") * _NC + lax.axis_index("c")
    row0 = wid * _ROWS_PER_W
    pltpu.sync_copy(seg_hbm.at[pl.ds(row0, _ROWS_PER_W)], seg_v)

    sval = lax.iota(jnp.int32, 16) + 1
    lo = jnp.zeros((16,), jnp.int32)
    hi = jnp.full((16,), _ROWS_PER_W, jnp.int32)
    for _ in range(11):
        active = lo < hi
        mid = jnp.minimum((lo + hi) // 2, _ROWS_PER_W - 1)
        below = plsc.load_gather(seg_v, [mid]) < sval
        lo = jnp.where(active & below, mid + 1, lo)
        hi = jnp.where(active & (~below), mid, hi)
    ub = lo

    bnd_v[pl.ds(0, 16)] = jnp.zeros((16,), jnp.int32)
    bnd_v[pl.ds(16, 16)] = ub
    ub_prev = bnd_v[pl.ds(15, 16)]
    cnt_v[pl.ds(0, 16)] = (ub - ub_prev).astype(jnp.float32)

    bounds = ([jnp.int32(0)] + [ub[s] for s in range(_NUM_SEG - 1)]
              + [jnp.int32(_ROWS_PER_W)])

    zero = jnp.zeros((16,), jnp.float32)
    for d in range(_NUM_SEG * _DIM // 16):
        acc_v[pl.ds(16 * d, 16)] = zero

    for c in range(_NCHUNK):
        c0 = c * _CHUNK
        pltpu.sync_copy(tree_hbm.at[pl.ds(row0 + c0, _CHUNK)], rows_v)
        for s in range(_NUM_SEG):
            start = jnp.clip(bounds[s] - c0, 0, _CHUNK)
            end = jnp.clip(bounds[s + 1] - c0, 0, _CHUNK)

            def body(r, accs):
                return tuple(a + rows_v[r, pl.ds(16 * d, 16)]
                             for d, a in enumerate(accs))

            a0, a1, a2, a3 = lax.fori_loop(start, end, body,
                                           (zero, zero, zero, zero))
            acc_v[pl.ds(s * _DIM, 16)] += a0
            acc_v[pl.ds(s * _DIM + 16, 16)] += a1
            acc_v[pl.ds(s * _DIM + 32, 16)] += a2
            acc_v[pl.ds(s * _DIM + 48, 16)] += a3

    pltpu.sync_copy(acc_v, sums_hbm.at[pl.ds(wid * _NUM_SEG * _DIM,
                                             _NUM_SEG * _DIM)])
    pltpu.sync_copy(cnt_v, counts_hbm.at[pl.ds(wid * _NUM_SEG, _NUM_SEG)])


def _mlp_body(sums_ref, counts_ref, w1_ref, b1_ref, w2t_ref, b2_ref, out_ref):
    totals = jnp.sum(sums_ref[...], axis=0)
    counts = jnp.sum(counts_ref[...], axis=0)
    agg = totals / jnp.maximum(counts, 1.0)[:, None]
    h = lax.dot_general(agg, w1_ref[...], (((1,), (0,)), ((), ())),
                        preferred_element_type=jnp.float32)
    h = jnp.maximum(h + b1_ref[...], 0.0)
    out_ref[...] = jnp.sum(h * w2t_ref[...], axis=1, keepdims=True) + b2_ref[...]


_mlp = pl.pallas_call(
    _mlp_body,
    out_shape=jax.ShapeDtypeStruct((_NUM_SEG, 1), jnp.float32),
)


def kernel(tree_h, W1, b1, W2, b2, segment_ids):
    seg = segment_ids.astype(jnp.int32)
    sums_flat, counts_flat = _sc_segment_sums(tree_h, seg)
    return _mlp(sums_flat.reshape(_NW, _NUM_SEG, _DIM),
                counts_flat.reshape(_NW, _NUM_SEG),
                W1, b1.reshape(1, _DIM), W2.reshape(1, _DIM),
                b2.reshape(1, 1))
